# baseline (speedup 1.0000x reference)
import jax
import jax.numpy as jnp
from jax.experimental import pallas as pl
from jax.experimental.pallas import tpu as pltpu


def _cc_kernel(cam_ref, wtab_ref, btab_ref, tex_ref, out_ref):
    b = pl.program_id(0)
    idx = cam_ref[b]
    for c in range(3):
        w = wtab_ref[idx, c]
        bb = btab_ref[idx, c]
        out_ref[0, c] = tex_ref[0, c] * w + bb


def kernel(texture, cam, weight, bias):
    B, C, H, W = texture.shape
    n_cam = weight.shape[0] + 1
    wtab = jnp.concatenate(
        [jnp.ones((1, C), texture.dtype), weight.reshape(n_cam - 1, C)], axis=0
    )
    btab = jnp.concatenate(
        [jnp.zeros((1, C), texture.dtype), bias.reshape(n_cam - 1, C)], axis=0
    )
    cam32 = cam.astype(jnp.int32)
    return pl.pallas_call(
        _cc_kernel,
        grid=(B,),
        in_specs=[
            pl.BlockSpec(memory_space=pltpu.SMEM),
            pl.BlockSpec(memory_space=pltpu.SMEM),
            pl.BlockSpec(memory_space=pltpu.SMEM),
            pl.BlockSpec((1, C, H, W), lambda b: (b, 0, 0, 0)),
        ],
        out_specs=pl.BlockSpec((1, C, H, W), lambda b: (b, 0, 0, 0)),
        out_shape=jax.ShapeDtypeStruct((B, C, H, W), texture.dtype),
    )(cam32, wtab, btab, texture)
